# baseline (speedup 1.0000x reference)
import functools

import jax
import jax.numpy as jnp
from jax import lax
from jax.experimental import pallas as pl
from jax.experimental.pallas import tpu as pltpu
from jax.experimental.pallas import tpu_sc as plsc

B = 4096
L = 200
E = 64
V = 1_000_000
NC = 2
NS = 16
NW = NC * NS
LANES = 16
VT_FULL = V // 128
V_TAIL = V - VT_FULL * 128
P_TAIL = VT_FULL * 64
BPW = B // NW

_params = pltpu.CompilerParams(
    use_tc_tiling_on_sc=True, needs_layout_passes=False
)


def _pack_table(tt, tail):
    mesh = plsc.VectorSubcoreMesh(core_axis_name="c", subcore_axis_name="s")

    @functools.partial(
        pl.kernel,
        mesh=mesh,
        out_type=jax.ShapeDtypeStruct((V // 2, 128), jnp.float32),
        compiler_params=_params,
        scratch_types=[
            pltpu.VMEM((E, 128), jnp.float32),
            pltpu.VMEM((E, 128), jnp.float32),
            pltpu.VMEM((E, 128), jnp.float32),
            pltpu.VMEM((E, 128), jnp.float32),
            pltpu.SemaphoreType.DMA,
            pltpu.SemaphoreType.DMA,
            pltpu.SemaphoreType.DMA,
            pltpu.SemaphoreType.DMA,
        ],
    )
    def k(tt_hbm, tail_hbm, out_hbm, s0, s1, t0, t1, sg0, sg1, sw0, sw1):
        wid = lax.axis_index("s") * NC + lax.axis_index("c")
        srcs = (s0, s1)
        dsts = (t0, t1)
        sg = (sg0, sg1)
        sw = (sw0, sw1)
        n_my = (VT_FULL - wid + NW - 1) // NW
        iota = lax.iota(jnp.int32, LANES)

        @pl.when(wid == 0)
        def _():
            pltpu.sync_copy(tail_hbm, out_hbm.at[pl.ds(P_TAIL, 32)])

        def fire_load(i, b):
            c = wid + i * NW
            pltpu.async_copy(
                tt_hbm.at[pl.ds(0, E), pl.ds(c * 128, 128)], srcs[b], sg[b]
            )

        def wait_load(b):
            pltpu.make_async_copy(
                tt_hbm.at[pl.ds(0, E), pl.ds(0, 128)], srcs[b], sg[b]
            ).wait()

        rots = [lax.rem(iota + r, 16) for r in range(16)]

        def transpose(b):
            src = srcs[b]
            dst = dsts[b]

            def jblock(jb, _):
                j0 = jb * 16
                for r0 in range(0, 16, 4):
                    meta = []
                    vals = []
                    for dr in range(4):
                        j_vec = j0 + rots[r0 + dr]
                        row_vec = lax.shift_right_logical(j_vec, 1)
                        col_base = lax.mul(lax.bitwise_and(j_vec, 1), 64) + iota
                        meta.append((row_vec, col_base))
                        for e0 in range(0, E, 16):
                            vals.append(plsc.load_gather(src, [e0 + iota, j_vec]))
                    for k, (dr, e0) in enumerate(
                        (dr, e0) for dr in range(4) for e0 in range(0, E, 16)
                    ):
                        row_vec, col_base = meta[dr]
                        plsc.store_scatter(dst, [row_vec, col_base + e0], vals[k])
                return 0

            lax.fori_loop(0, 8, jblock, 0)

        def fire_store(i, b):
            c = wid + i * NW
            pltpu.async_copy(dsts[b], out_hbm.at[pl.ds(c * 64, 64)], sw[b])

        def wait_store(b):
            pltpu.make_async_copy(
                dsts[b], out_hbm.at[pl.ds(0, 64)], sw[b]
            ).wait()

        def step(i, b):
            @pl.when(i + 1 < n_my)
            def _():
                fire_load(i + 1, 1 - b)

            wait_load(b)

            @pl.when(i >= 2)
            def _():
                wait_store(b)

            transpose(b)
            fire_store(i, b)

        fire_load(0, 0)

        def loop_body(i, _):
            @pl.when(lax.rem(i, 2) == 0)
            def _():
                step(i, 0)

            @pl.when(lax.rem(i, 2) == 1)
            def _():
                step(i, 1)

            return 0

        lax.fori_loop(0, n_my, loop_body, 0)
        wait_store(0)
        wait_store(1)

    return k(tt, tail)


def _lookup(packed, xt, mt):
    mesh = plsc.VectorSubcoreMesh(core_axis_name="c", subcore_axis_name="s")

    @functools.partial(
        pl.kernel,
        mesh=mesh,
        out_type=jax.ShapeDtypeStruct((L, E, B), jnp.float32),
        compiler_params=_params,
        scratch_types=[
            pltpu.VMEM((L, BPW), jnp.int32),
            pltpu.VMEM((L, BPW), jnp.float32),
            pltpu.VMEM((BPW,), jnp.int32),
            pltpu.VMEM((BPW,), jnp.int32),
            pltpu.VMEM((BPW, 128), jnp.float32),
            pltpu.VMEM((BPW, 128), jnp.float32),
            pltpu.VMEM((E, BPW), jnp.float32),
            pltpu.VMEM((E, BPW), jnp.float32),
            pltpu.SemaphoreType.DMA,
            pltpu.SemaphoreType.DMA,
            pltpu.SemaphoreType.DMA,
            pltpu.SemaphoreType.DMA,
        ],
    )
    def k(packed_hbm, xt_hbm, mt_hbm, out_hbm,
          xv, mv, p0, p1, g0, g1, t0, t1, sg0, sg1, sw0, sw1):
        wid = lax.axis_index("s") * NC + lax.axis_index("c")
        b0 = wid * BPW
        pids = (p0, p1)
        gbuf = (g0, g1)
        tbuf = (t0, t1)
        sg = (sg0, sg1)
        sw = (sw0, sw1)
        iota = lax.iota(jnp.int32, LANES)

        pltpu.sync_copy(xt_hbm.at[pl.ds(0, L), pl.ds(b0, BPW)], xv)
        pltpu.sync_copy(mt_hbm.at[pl.ds(0, L), pl.ds(b0, BPW)], mv)

        def fire_gather(l, b):
            for g in range(8):
                idx = xv[l, pl.ds(g * 16, 16)]
                pids[b][pl.ds(g * 16, 16)] = lax.shift_right_logical(idx, 1)
            pltpu.async_copy(packed_hbm.at[pids[b]], gbuf[b], sg[b])

        def wait_gather(b):
            pltpu.make_async_copy(
                packed_hbm.at[pl.ds(0, BPW)], gbuf[b], sg[b]
            ).wait()

        rots = [lax.rem(iota + r, 16) for r in range(16)]

        def transpose(l, b):
            src = gbuf[b]
            dst = tbuf[b]

            def gblock(g, _):
                tok = g * 16 + iota
                idx = xv[l, pl.ds(g * 16, 16)]
                off = lax.mul(lax.bitwise_and(idx, 1), 64)
                msk = mv[l, pl.ds(g * 16, 16)]
                for r0 in range(0, 16, 4):
                    vals = []
                    for dr in range(4):
                        e_rot = rots[r0 + dr]
                        for e0 in range(0, E, 16):
                            v = plsc.load_gather(src, [tok, off + (e0 + e_rot)])
                            vals.append((e0 + e_rot, v))
                    for e_vec, v in vals:
                        plsc.store_scatter(dst, [e_vec, tok], v * msk)
                return 0

            lax.fori_loop(0, 8, gblock, 0)

        def fire_write(l, b):
            for t in range(8):
                pltpu.async_copy(
                    tbuf[b].at[pl.ds(t * 8, 8)],
                    out_hbm.at[l, pl.ds(t * 8, 8), pl.ds(b0, BPW)],
                    sw[b],
                )

        def wait_writes(b):
            pltpu.make_async_copy(
                tbuf[b], out_hbm.at[0, pl.ds(0, E), pl.ds(0, BPW)], sw[b]
            ).wait()

        fire_gather(0, 0)
        wait_gather(0)
        fire_gather(1, 1)
        transpose(0, 0)
        fire_write(0, 0)
        wait_gather(1)
        fire_gather(2, 0)
        transpose(1, 1)
        fire_write(1, 1)

        def pair_body(q, _):
            l = 2 + 2 * q
            wait_gather(0)
            fire_gather(l + 1, 1)
            wait_writes(0)
            transpose(l, 0)
            fire_write(l, 0)
            wait_gather(1)
            fire_gather(l + 2, 0)
            wait_writes(1)
            transpose(l + 1, 1)
            fire_write(l + 1, 1)
            return 0

        lax.fori_loop(0, (L - 4) // 2, pair_body, 0)
        l = L - 2
        wait_gather(0)
        fire_gather(l + 1, 1)
        wait_writes(0)
        transpose(l, 0)
        fire_write(l, 0)
        wait_gather(1)
        wait_writes(1)
        transpose(l + 1, 1)
        fire_write(l + 1, 1)
        wait_writes(0)
        wait_writes(1)

    return k(packed, xt, mt)


def kernel(x, attention_mask, table):
    tt = table.T
    xt = x.T
    mt = attention_mask.T
    tail = table[VT_FULL * 128:, :].reshape(32, 128)
    packed = _pack_table(tt, tail)
    out3 = _lookup(packed, xt, mt)
    return jnp.transpose(out3, (2, 0, 1))


# speedup vs baseline: 1.0411x; 1.0411x over previous
import functools

import jax
import jax.numpy as jnp
from jax import lax
from jax.experimental import pallas as pl
from jax.experimental.pallas import tpu as pltpu
from jax.experimental.pallas import tpu_sc as plsc

B = 4096
L = 200
E = 64
V = 1_000_000
NC = 2
NS = 16
NW = NC * NS
LANES = 16
VT_FULL = V // 128
V_TAIL = V - VT_FULL * 128
P_TAIL = VT_FULL * 64
BPW = B // NW

_params = pltpu.CompilerParams(
    use_tc_tiling_on_sc=True, needs_layout_passes=False
)


def _pack_table(tt, tail):
    mesh = plsc.VectorSubcoreMesh(core_axis_name="c", subcore_axis_name="s")

    @functools.partial(
        pl.kernel,
        mesh=mesh,
        out_type=jax.ShapeDtypeStruct((V // 2, 128), jnp.float32),
        compiler_params=_params,
        scratch_types=[
            pltpu.VMEM((E, 128), jnp.float32),
            pltpu.VMEM((E, 128), jnp.float32),
            pltpu.VMEM((E, 128), jnp.float32),
            pltpu.VMEM((E, 128), jnp.float32),
            pltpu.SemaphoreType.DMA,
            pltpu.SemaphoreType.DMA,
            pltpu.SemaphoreType.DMA,
            pltpu.SemaphoreType.DMA,
        ],
    )
    def k(tt_hbm, tail_hbm, out_hbm, s0, s1, t0, t1, sg0, sg1, sw0, sw1):
        wid = lax.axis_index("s") * NC + lax.axis_index("c")
        srcs = (s0, s1)
        dsts = (t0, t1)
        sg = (sg0, sg1)
        sw = (sw0, sw1)
        n_my = (VT_FULL - wid + NW - 1) // NW
        iota = lax.iota(jnp.int32, LANES)

        @pl.when(wid == 0)
        def _():
            pltpu.sync_copy(tail_hbm, out_hbm.at[pl.ds(P_TAIL, 32)])

        def fire_load(i, b):
            c = wid + i * NW
            pltpu.async_copy(
                tt_hbm.at[pl.ds(0, E), pl.ds(c * 128, 128)], srcs[b], sg[b]
            )

        def wait_load(b):
            pltpu.make_async_copy(
                tt_hbm.at[pl.ds(0, E), pl.ds(0, 128)], srcs[b], sg[b]
            ).wait()

        rots = [lax.rem(iota + r, 16) for r in range(16)]

        def transpose(b):
            src = srcs[b]
            dst = dsts[b]

            def jblock(jb, _):
                j0 = jb * 16
                for r0 in range(0, 16, 2):
                    meta = []
                    vals = []
                    for dr in range(2):
                        j_vec = j0 + rots[r0 + dr]
                        row_vec = lax.shift_right_logical(j_vec, 1)
                        col_base = lax.mul(lax.bitwise_and(j_vec, 1), 64) + iota
                        meta.append((row_vec, col_base))
                        for e0 in range(0, E, 16):
                            vals.append(plsc.load_gather(src, [e0 + iota, j_vec]))
                    for k, (dr, e0) in enumerate(
                        (dr, e0) for dr in range(2) for e0 in range(0, E, 16)
                    ):
                        row_vec, col_base = meta[dr]
                        plsc.store_scatter(dst, [row_vec, col_base + e0], vals[k])
                return 0

            lax.fori_loop(0, 8, jblock, 0)

        def fire_store(i, b):
            c = wid + i * NW
            pltpu.async_copy(dsts[b], out_hbm.at[pl.ds(c * 64, 64)], sw[b])

        def wait_store(b):
            pltpu.make_async_copy(
                dsts[b], out_hbm.at[pl.ds(0, 64)], sw[b]
            ).wait()

        def step(i, b):
            @pl.when(i + 1 < n_my)
            def _():
                fire_load(i + 1, 1 - b)

            wait_load(b)

            @pl.when(i >= 2)
            def _():
                wait_store(b)

            transpose(b)
            fire_store(i, b)

        fire_load(0, 0)

        def loop_body(i, _):
            @pl.when(lax.rem(i, 2) == 0)
            def _():
                step(i, 0)

            @pl.when(lax.rem(i, 2) == 1)
            def _():
                step(i, 1)

            return 0

        lax.fori_loop(0, n_my, loop_body, 0)
        wait_store(0)
        wait_store(1)

    return k(tt, tail)


def _lookup(packed, xt, mt):
    mesh = plsc.VectorSubcoreMesh(core_axis_name="c", subcore_axis_name="s")

    @functools.partial(
        pl.kernel,
        mesh=mesh,
        out_type=jax.ShapeDtypeStruct((L, E, B), jnp.float32),
        compiler_params=_params,
        scratch_types=[
            pltpu.VMEM((L, BPW), jnp.int32),
            pltpu.VMEM((L, BPW), jnp.float32),
            pltpu.VMEM((BPW,), jnp.int32),
            pltpu.VMEM((BPW,), jnp.int32),
            pltpu.VMEM((BPW, 128), jnp.float32),
            pltpu.VMEM((BPW, 128), jnp.float32),
            pltpu.VMEM((E, BPW), jnp.float32),
            pltpu.VMEM((E, BPW), jnp.float32),
            pltpu.SemaphoreType.DMA,
            pltpu.SemaphoreType.DMA,
            pltpu.SemaphoreType.DMA,
            pltpu.SemaphoreType.DMA,
        ],
    )
    def k(packed_hbm, xt_hbm, mt_hbm, out_hbm,
          xv, mv, p0, p1, g0, g1, t0, t1, sg0, sg1, sw0, sw1):
        wid = lax.axis_index("s") * NC + lax.axis_index("c")
        b0 = wid * BPW
        pids = (p0, p1)
        gbuf = (g0, g1)
        tbuf = (t0, t1)
        sg = (sg0, sg1)
        sw = (sw0, sw1)
        iota = lax.iota(jnp.int32, LANES)

        pltpu.sync_copy(xt_hbm.at[pl.ds(0, L), pl.ds(b0, BPW)], xv)
        pltpu.sync_copy(mt_hbm.at[pl.ds(0, L), pl.ds(b0, BPW)], mv)

        def fire_gather(l, b):
            for g in range(8):
                idx = xv[l, pl.ds(g * 16, 16)]
                pids[b][pl.ds(g * 16, 16)] = lax.shift_right_logical(idx, 1)
            pltpu.async_copy(packed_hbm.at[pids[b]], gbuf[b], sg[b])

        def wait_gather(b):
            pltpu.make_async_copy(
                packed_hbm.at[pl.ds(0, BPW)], gbuf[b], sg[b]
            ).wait()

        rots = [lax.rem(iota + r, 16) for r in range(16)]

        def transpose(l, b):
            src = gbuf[b]
            dst = tbuf[b]

            def gblock(g, _):
                tok = g * 16 + iota
                idx = xv[l, pl.ds(g * 16, 16)]
                off = lax.mul(lax.bitwise_and(idx, 1), 64)
                msk = mv[l, pl.ds(g * 16, 16)]
                for r0 in range(0, 16, 2):
                    vals = []
                    for dr in range(2):
                        e_rot = rots[r0 + dr]
                        for e0 in range(0, E, 16):
                            v = plsc.load_gather(src, [tok, off + (e0 + e_rot)])
                            vals.append((e0 + e_rot, v))
                    for e_vec, v in vals:
                        plsc.store_scatter(dst, [e_vec, tok], v * msk)
                return 0

            lax.fori_loop(0, 8, gblock, 0)

        def fire_write(l, b):
            for t in range(8):
                pltpu.async_copy(
                    tbuf[b].at[pl.ds(t * 8, 8)],
                    out_hbm.at[l, pl.ds(t * 8, 8), pl.ds(b0, BPW)],
                    sw[b],
                )

        def wait_writes(b):
            pltpu.make_async_copy(
                tbuf[b], out_hbm.at[0, pl.ds(0, E), pl.ds(0, BPW)], sw[b]
            ).wait()

        fire_gather(0, 0)
        wait_gather(0)
        fire_gather(1, 1)
        transpose(0, 0)
        fire_write(0, 0)
        wait_gather(1)
        fire_gather(2, 0)
        transpose(1, 1)
        fire_write(1, 1)

        def pair_body(q, _):
            l = 2 + 2 * q
            wait_gather(0)
            fire_gather(l + 1, 1)
            wait_writes(0)
            transpose(l, 0)
            fire_write(l, 0)
            wait_gather(1)
            fire_gather(l + 2, 0)
            wait_writes(1)
            transpose(l + 1, 1)
            fire_write(l + 1, 1)
            return 0

        lax.fori_loop(0, (L - 4) // 2, pair_body, 0)
        l = L - 2
        wait_gather(0)
        fire_gather(l + 1, 1)
        wait_writes(0)
        transpose(l, 0)
        fire_write(l, 0)
        wait_gather(1)
        wait_writes(1)
        transpose(l + 1, 1)
        fire_write(l + 1, 1)
        wait_writes(0)
        wait_writes(1)

    return k(packed, xt, mt)


def kernel(x, attention_mask, table):
    tt = table.T
    xt = x.T
    mt = attention_mask.T
    tail = table[VT_FULL * 128:, :].reshape(32, 128)
    packed = _pack_table(tt, tail)
    out3 = _lookup(packed, xt, mt)
    return jnp.transpose(out3, (2, 0, 1))
